# baseline (speedup 1.0000x reference)
import functools
import math

import jax
import jax.numpy as jnp
from jax import lax
from jax.experimental import pallas as pl
from jax.experimental.pallas import tpu as pltpu
from jax.experimental.pallas import tpu_sc as plsc

EMB = 128
SCALE = math.sqrt(EMB)

_info = plsc.get_sparse_core_info()
_NC, _NS = _info.num_cores, _info.num_subcores
_NW = _NC * _NS

CHUNK = 128


def _scale_body(t_ref, o_ref):
    o_ref[...] = t_ref[...] * SCALE


def _scale_table(table):
    rows = table.shape[0]
    block = 2000
    assert rows % block == 0
    return pl.pallas_call(
        _scale_body,
        out_shape=jax.ShapeDtypeStruct(table.shape, table.dtype),
        grid=(rows // block,),
        in_specs=[pl.BlockSpec((block, EMB), lambda i: (i, 0))],
        out_specs=pl.BlockSpec((block, EMB), lambda i: (i, 0)),
    )(table)


@functools.lru_cache(maxsize=None)
def _make_gather(n_chunk_rows):
    rows_per_w = n_chunk_rows // _NW
    mesh = plsc.VectorSubcoreMesh(core_axis_name="c", subcore_axis_name="s")

    nbuf = 6
    d = nbuf // 2
    n = rows_per_w
    ngroup = n // nbuf
    main = ngroup * nbuf

    @functools.partial(
        pl.kernel,
        out_type=jax.ShapeDtypeStruct((n_chunk_rows * CHUNK, EMB), jnp.float32),
        mesh=mesh,
        scratch_types=[
            pltpu.VMEM((rows_per_w, CHUNK), jnp.int32),
            [pltpu.VMEM((CHUNK, EMB), jnp.float32) for _ in range(nbuf)],
            [pltpu.SemaphoreType.DMA for _ in range(nbuf)],
            [pltpu.SemaphoreType.DMA for _ in range(nbuf)],
        ],
    )
    def k(table_hbm, tok_hbm, out_hbm, idx_v, bufs, gsems, osems):
        wid = lax.axis_index("s") * _NC + lax.axis_index("c")
        base = wid * rows_per_w
        pltpu.sync_copy(tok_hbm.at[pl.ds(base, rows_per_w)], idx_v)

        def gather(j, b):
            return pltpu.make_async_copy(
                table_hbm.at[idx_v.at[j]], bufs[b], gsems[b])

        def outcp(j, b):
            return pltpu.make_async_copy(
                bufs[b], out_hbm.at[pl.ds((base + j) * CHUNK, CHUNK)], osems[b])

        def scale_buf(b):
            @plsc.parallel_loop(0, CHUNK, 1, unroll=4)
            def _(r):
                for c in range(EMB // 16):
                    sl = (r, pl.ds(c * 16, 16))
                    bufs[b][sl] = bufs[b][sl] * SCALE

        for j in range(d):
            gather(j, j % nbuf).start()

        def body(g, carry):
            for b in range(nbuf):
                j = g * nbuf + b
                gather(j, b).wait()
                bw = (b - d) % nbuf
                if b >= d:
                    outcp(j - d, bw).wait()
                else:
                    @pl.when(g > 0)
                    def _():
                        outcp(j - d, bw).wait()
                bg = (b + d) % nbuf
                if (ngroup - 1) * nbuf + b + d <= n - 1:
                    gather(j + d, bg).start()
                else:
                    @pl.when(j + d < n)
                    def _():
                        gather(j + d, bg).start()
                scale_buf(b)
                outcp(j, b).start()
            return carry

        lax.fori_loop(0, ngroup, body, 0)

        for j in range(main, n):
            b = j % nbuf
            gather(j, b).wait()
            outcp(j - d, (j - d) % nbuf).wait()
            scale_buf(b)
            outcp(j, b).start()
        for j in range(n - d, n):
            outcp(j, j % nbuf).wait()

    return k


def kernel(tokens, table):
    b, t = tokens.shape
    flat = b * t
    assert flat % (_NW * CHUNK) == 0
    tok2d = tokens.astype(jnp.int32).reshape(flat // CHUNK, CHUNK)
    scaled = table
    out = _make_gather(flat // CHUNK)(scaled, tok2d)
    return out.reshape(b, t, EMB)


# speedup vs baseline: 1.0027x; 1.0027x over previous
import functools
import math

import jax
import jax.numpy as jnp
from jax import lax
from jax.experimental import pallas as pl
from jax.experimental.pallas import tpu as pltpu
from jax.experimental.pallas import tpu_sc as plsc

EMB = 128
SCALE = math.sqrt(EMB)

_info = plsc.get_sparse_core_info()
_NC, _NS = _info.num_cores, _info.num_subcores
_NW = _NC * _NS

CHUNK = 128


@functools.lru_cache(maxsize=None)
def _make_gather(n_chunk_rows):
    rows_per_w = n_chunk_rows // _NW
    mesh = plsc.VectorSubcoreMesh(core_axis_name="c", subcore_axis_name="s")

    nbuf = 6
    d = nbuf // 2
    n = rows_per_w
    ngroup = n // nbuf
    main = ngroup * nbuf

    @functools.partial(
        pl.kernel,
        out_type=jax.ShapeDtypeStruct((n_chunk_rows * CHUNK, EMB), jnp.float32),
        mesh=mesh,
        scratch_types=[
            pltpu.VMEM((rows_per_w, CHUNK), jnp.int32),
            [pltpu.VMEM((CHUNK, EMB), jnp.float32) for _ in range(nbuf)],
            [pltpu.SemaphoreType.DMA for _ in range(nbuf)],
            [pltpu.SemaphoreType.DMA for _ in range(nbuf)],
        ],
    )
    def k(table_hbm, tok_hbm, out_hbm, idx_v, bufs, gsems, osems):
        wid = lax.axis_index("s") * _NC + lax.axis_index("c")
        base = wid * rows_per_w
        pltpu.sync_copy(tok_hbm.at[pl.ds(base, rows_per_w)], idx_v)

        def gather(j, b):
            return pltpu.make_async_copy(
                table_hbm.at[idx_v.at[j]], bufs[b], gsems[b])

        def outcp(j, b):
            return pltpu.make_async_copy(
                bufs[b], out_hbm.at[pl.ds((base + j) * CHUNK, CHUNK)], osems[b])

        def scale_buf(b):
            @plsc.parallel_loop(0, CHUNK, 1, unroll=4)
            def _(r):
                for c in range(EMB // 16):
                    sl = (r, pl.ds(c * 16, 16))
                    bufs[b][sl] = bufs[b][sl] * SCALE

        for j in range(d):
            gather(j, j % nbuf).start()

        def body(g, carry):
            for b in range(nbuf):
                j = g * nbuf + b
                gather(j, b).wait()
                bw = (b - d) % nbuf
                if b >= d:
                    outcp(j - d, bw).wait()
                else:
                    @pl.when(g > 0)
                    def _():
                        outcp(j - d, bw).wait()
                bg = (b + d) % nbuf
                if (ngroup - 1) * nbuf + b + d <= n - 1:
                    gather(j + d, bg).start()
                else:
                    @pl.when(j + d < n)
                    def _():
                        gather(j + d, bg).start()
                scale_buf(b)
                outcp(j, b).start()
            return carry

        lax.fori_loop(0, ngroup, body, 0)

        for j in range(main, n):
            b = j % nbuf
            gather(j, b).wait()
            outcp(j - d, (j - d) % nbuf).wait()
            scale_buf(b)
            outcp(j, b).start()
        for j in range(n - d, n):
            outcp(j, j % nbuf).wait()

    return k


def kernel(tokens, table):
    b, t = tokens.shape
    flat = b * t
    assert flat % (_NW * CHUNK) == 0
    tok2d = tokens.astype(jnp.int32).reshape(flat // CHUNK, CHUNK)
    out = _make_gather(flat // CHUNK)(table, tok2d)
    return out.reshape(b, t, EMB)
